# baseline (speedup 1.0000x reference)
import functools

import jax
import jax.numpy as jnp
from jax import lax
from jax.experimental import pallas as pl
from jax.experimental.pallas import tpu as pltpu
from jax.experimental.pallas import tpu_sc as plsc

B_ = 1
S = 8192
D = 768
H = 3072
E = 64
C = 128
NC, NS, L = 2, 16, 16
TB = 512
ONE_F32_BITS = 0x3F800000


def _router_body(x_ref, wg_ref, probs_ref, thr_ref):
    i = pl.program_id(0)

    @pl.when(i < S // TB)
    def _():
        xb = x_ref[...]
        wg = wg_ref[...]
        logits = lax.dot_general(wg, xb, (((0,), (1,)), ((), ())),
                                 preferred_element_type=jnp.float32)
        m = jnp.max(logits, axis=0, keepdims=True)
        p = jnp.exp(logits - m)
        probs_ref[:, pl.ds(i * TB, TB)] = p / jnp.sum(p, axis=0, keepdims=True)

    @pl.when(i == S // TB)
    def _():
        bits = pltpu.bitcast(probs_ref[...], jnp.int32)

        def step(_, lohi):
            lo, hi = lohi
            mid = (lo + hi + 1) >> 1
            cnt = jnp.sum((bits >= mid).astype(jnp.int32), axis=1,
                          keepdims=True)
            ok = cnt >= C
            return jnp.where(ok, mid, lo), jnp.where(ok, hi, mid - 1)

        lo = jnp.zeros((E, 1), jnp.int32)
        hi = jnp.full((E, 1), ONE_F32_BITS, jnp.int32)
        lo, _ = lax.fori_loop(0, 31, step, (lo, hi))
        thr_ref[...] = pltpu.bitcast(jnp.broadcast_to(lo, (E, L)), jnp.float32)


def _router(x2d, Wg):
    return pl.pallas_call(
        _router_body,
        grid=(S // TB + 1,),
        in_specs=[
            pl.BlockSpec((TB, D), lambda i: (jnp.minimum(i, S // TB - 1), 0)),
            pl.BlockSpec((D, E), lambda i: (0, 0)),
        ],
        out_specs=[
            pl.BlockSpec((E, S), lambda i: (0, 0)),
            pl.BlockSpec((E, L), lambda i: (0, 0)),
        ],
        out_shape=[
            jax.ShapeDtypeStruct((E, S), jnp.float32),
            jax.ShapeDtypeStruct((E, L), jnp.float32),
        ],
    )(x2d, Wg)


def _select_gather(probsT, thr, x2d):
    mesh = plsc.VectorSubcoreMesh(core_axis_name="c", subcore_axis_name="s")
    rows_per_worker = E // (NC * NS)

    @functools.partial(
        pl.kernel,
        out_type=[
            jax.ShapeDtypeStruct((E, C), jnp.int32),
            jax.ShapeDtypeStruct((E, C), jnp.float32),
            jax.ShapeDtypeStruct((S, D), jnp.float32),
        ],
        mesh=mesh,
        compiler_params=pltpu.CompilerParams(needs_layout_passes=False),
        scratch_types=[
            pltpu.VMEM((S,), jnp.float32),
            pltpu.VMEM((L,), jnp.float32),
            pltpu.VMEM((C,), jnp.int32),
            pltpu.VMEM((C,), jnp.float32),
            pltpu.VMEM((C, D), jnp.float32),
            pltpu.VMEM((S // L,), jnp.int32),
            pltpu.VMEM((S // L,), jnp.int32),
            pltpu.VMEM((S // L,), jnp.int32),
        ],
    )
    def k(probs_hbm, thr_hbm, x_hbm, idx_out, sc_out, tok_out,
          pr_v, thr_v, idx_v, sc_v, rows_v, cnt_v, pfx_v, nz_v):
        wid = lax.axis_index("s") * NC + lax.axis_index("c")
        iota = lax.iota(jnp.int32, L)
        zero = jnp.zeros((L,), jnp.int32)
        lane0 = iota == 0
        for r in range(rows_per_worker):
            e = wid * rows_per_worker + r
            pltpu.sync_copy(probs_hbm.at[e], pr_v)
            pltpu.sync_copy(thr_hbm.at[e], thr_v)
            thr_vec = thr_v[...]

            def p1(j, _):
                v = pr_v[pl.ds(j * L, L)]
                cntv = plsc.all_reduce_population_count(v >= thr_vec)
                plsc.store_scatter(cnt_v, [zero + j], cntv, mask=lane0)
                return 0

            lax.fori_loop(0, S // L, p1, 0)

            def p2(jb, carry):
                off, nz = carry
                c16 = cnt_v[pl.ds(jb * L, L)]
                cum = plsc.cumsum(c16)
                pfx_v[pl.ds(jb * L, L)] = off + cum - c16
                nzi = (c16 > 0).astype(jnp.int32)
                nzc = jnp.sum(nzi)

                @pl.when(nzc > 0)
                def _():
                    posz = nz + plsc.cumsum(nzi) - 1
                    plsc.store_scatter(nz_v, [posz], iota + jb * L,
                                       mask=c16 > 0)

                return off + jnp.sum(c16), nz + nzc

            _, nnz = lax.fori_loop(0, S // (L * L), p2,
                                   (jnp.int32(0), jnp.int32(0)))

            def p3(m, _):
                jsp = plsc.load_gather(nz_v, [zero + m])
                lanes = jsp * L + iota
                v = plsc.load_gather(pr_v, [lanes])
                ge = v >= thr_vec
                base = plsc.load_gather(pfx_v, [jsp])
                pos = base + plsc.cumsum(ge.astype(jnp.int32)) - 1
                msk = ge & (pos < C)
                plsc.store_scatter(idx_v, [pos], lanes, mask=msk)
                plsc.store_scatter(sc_v, [pos], v, mask=msk)
                return 0

            lax.fori_loop(0, nnz, p3, 0)
            pltpu.sync_copy(x_hbm.at[idx_v], rows_v)
            pltpu.sync_copy(rows_v, tok_out.at[pl.ds(e * C, C)])
            pltpu.sync_copy(idx_v, idx_out.at[e])
            pltpu.sync_copy(sc_v, sc_out.at[e])

    return k(probsT, thr, x2d)


def _ffn_body(tok_ref, w1_ref, b1_ref, w2_ref, b2_ref, sc_ref,
              x_ref, ws1_ref, bs1_ref, ws2_ref, bs2_ref, eo_ref, sh_ref):
    bf = jnp.bfloat16
    tok = tok_ref[...].astype(bf)
    h = jnp.dot(tok, w1_ref[0].astype(bf), preferred_element_type=jnp.float32)
    h = jax.nn.gelu(h + b1_ref[0], approximate=True)
    o = jnp.dot(h.astype(bf), w2_ref[0].astype(bf),
                preferred_element_type=jnp.float32)
    eo_ref[...] = ((o + b2_ref[0]) * sc_ref[...]).astype(bf)

    xb = x_ref[...].astype(bf)
    hs = jnp.dot(xb, ws1_ref[...], preferred_element_type=jnp.float32)
    hs = jax.nn.gelu(hs + bs1_ref[...], approximate=True)
    sh_ref[...] = jnp.dot(hs.astype(bf), ws2_ref[...],
                          preferred_element_type=jnp.float32) + bs2_ref[...]


def _ffn(toks, W1, b1, W2, b2, scores_col, x2d, Ws1b, bs1_2d, Ws2b, bs2_2d):
    return pl.pallas_call(
        _ffn_body,
        grid=(E,),
        in_specs=[
            pl.BlockSpec((C, D), lambda e: (e, 0)),
            pl.BlockSpec((1, D, H), lambda e: (e, 0, 0)),
            pl.BlockSpec((1, 1, H), lambda e: (e, 0, 0)),
            pl.BlockSpec((1, H, D), lambda e: (e, 0, 0)),
            pl.BlockSpec((1, 1, D), lambda e: (e, 0, 0)),
            pl.BlockSpec((C, 1), lambda e: (e, 0)),
            pl.BlockSpec((C, D), lambda e: (e, 0)),
            pl.BlockSpec((D, H), lambda e: (0, 0)),
            pl.BlockSpec((1, H), lambda e: (0, 0)),
            pl.BlockSpec((H, D), lambda e: (0, 0)),
            pl.BlockSpec((1, D), lambda e: (0, 0)),
        ],
        out_specs=[
            pl.BlockSpec((C, D), lambda e: (e, 0)),
            pl.BlockSpec((C, D), lambda e: (e, 0)),
        ],
        out_shape=[
            jax.ShapeDtypeStruct((S, D), jnp.bfloat16),
            jax.ShapeDtypeStruct((S, D), jnp.float32),
        ],
        compiler_params=pltpu.CompilerParams(
            vmem_limit_bytes=112 * 1024 * 1024),
    )(toks, W1, b1, W2, b2, scores_col, x2d, Ws1b, bs1_2d, Ws2b, bs2_2d)


TE = 1024


def _combine_body(idx_ref, eo_ref, sh_ref, out_ref):
    ids = idx_ref[...]
    t0 = pl.program_id(0) * TE
    tok = jax.lax.broadcasted_iota(jnp.int32, (1, TE), 1) + t0
    onehot = (ids == tok).astype(jnp.bfloat16)
    acc = lax.dot_general(onehot, eo_ref[...], (((0,), (0,)), ((), ())),
                          preferred_element_type=jnp.float32)
    out_ref[...] = acc + sh_ref[...]


def _combine(eo_bf, idx_col, sh):
    return pl.pallas_call(
        _combine_body,
        grid=(S // TE,),
        in_specs=[
            pl.BlockSpec((S, 1), lambda t: (0, 0)),
            pl.BlockSpec((S, D), lambda t: (0, 0)),
            pl.BlockSpec((TE, D), lambda t: (t, 0)),
        ],
        out_specs=pl.BlockSpec((TE, D), lambda t: (t, 0)),
        out_shape=jax.ShapeDtypeStruct((S, D), jnp.float32),
    )(idx_col, eo_bf, sh)


def kernel(x, Wg, W1, b1, W2, b2, Ws1, bs1, Ws2, bs2):
    x2d = x.reshape(S, D)
    probsT, thr = _router(x2d, Wg)
    idx, scores, toks = _select_gather(probsT, thr, x2d)
    eo, sh = _ffn(toks, W1, b1.reshape(E, 1, H), W2, b2.reshape(E, 1, D),
                  scores.reshape(S, 1), x2d,
                  Ws1.astype(jnp.bfloat16), bs1.reshape(1, H),
                  Ws2.astype(jnp.bfloat16), bs2.reshape(1, D))
    out = _combine(eo, idx.reshape(S, 1), sh)
    return out.reshape(B_, S, D)
